# baseline (speedup 1.0000x reference)
import functools

import jax
import jax.numpy as jnp
from jax import lax
from jax.experimental import pallas as pl
from jax.experimental.pallas import tpu as pltpu
from jax.experimental.pallas import tpu_sc as plsc

EMB_DIM = 64
SCALE = 8.0

NUM_CORES = 2
NUM_SUBCORES = 16
NUM_WORKERS = NUM_CORES * NUM_SUBCORES

CH_OUT = 64
CH_LOOK = 2 * CH_OUT
NBUF = 4


def _make_gather(n_out2):
    assert n_out2 % (NUM_WORKERS * CH_OUT * NBUF) == 0
    o_per_w = n_out2 // NUM_WORKERS
    n_chunks = o_per_w // CH_OUT
    mesh = plsc.VectorSubcoreMesh(core_axis_name="c", subcore_axis_name="s")

    @functools.partial(
        pl.kernel,
        mesh=mesh,
        out_type=jax.ShapeDtypeStruct((n_out2, 128), jnp.float32),
        scratch_types=[
            pltpu.VMEM((NBUF, CH_LOOK), jnp.int32),
            pltpu.VMEM((NBUF, CH_LOOK), jnp.int32),
            pltpu.VMEM((NBUF, CH_LOOK, 128), jnp.float32),
            pltpu.VMEM((NBUF, CH_OUT, 128), jnp.float32),
            [pltpu.SemaphoreType.DMA] * NBUF,
            [pltpu.SemaphoreType.DMA] * NBUF,
        ],
        compiler_params=pltpu.CompilerParams(use_tc_tiling_on_sc=True),
    )
    def gather_kernel(t2_hbm, idx_hbm, out_hbm, idx_v, idx2_v, g_v, o_v,
                      gsems, ssems):
        wid = lax.axis_index("s") * NUM_CORES + lax.axis_index("c")
        base = wid * o_per_w

        def start_gather(s, off):
            pltpu.sync_copy(idx_hbm.at[pl.ds(2 * off, CH_LOOK)], idx_v.at[s])
            for q in range(CH_LOOK // 16):
                sl = pl.ds(16 * q, 16)
                idx2_v.at[s][sl] = lax.shift_right_logical(
                    idx_v.at[s][sl], jnp.int32(1))
            pltpu.make_async_copy(
                t2_hbm.at[idx2_v.at[s]], g_v.at[s], gsems[s]
            ).start()

        def pack(s):
            def pack_body(q, _):
                pv = idx_v.at[s][pl.ds(16 * q, 16)] & jnp.int32(1)
                for r in range(8):
                    j = 8 * q + r
                    for h in range(2):
                        lane = 2 * r + h
                        par = pv.at[jnp.full((16,), lane, jnp.int32)].get(
                            mode="promise_in_bounds")
                        pf = par.astype(jnp.float32)
                        row = g_v.at[s].at[16 * q + lane]
                        for k in range(4):
                            a = row[pl.ds(16 * k, 16)]
                            b = row[pl.ds(64 + 16 * k, 16)]
                            o_v.at[s][j, pl.ds(64 * h + 16 * k, 16)] = (
                                a + (b - a) * pf) * SCALE
                return 0

            lax.fori_loop(0, CH_LOOK // 16, pack_body, 0, unroll=False)

        def process(s, off, prefetch):
            pltpu.make_async_copy(
                t2_hbm.at[idx2_v.at[s]], g_v.at[s], gsems[s]
            ).wait()
            pack(s)
            pltpu.make_async_copy(
                o_v.at[s], out_hbm.at[pl.ds(off, CH_OUT)], ssems[s]
            ).start()
            pltpu.make_async_copy(
                o_v.at[s], out_hbm.at[pl.ds(off, CH_OUT)], ssems[s]
            ).wait()
            if prefetch:
                start_gather(s, off + NBUF * CH_OUT)

        for s in range(NBUF):
            start_gather(s, base + s * CH_OUT)

        def body(i, _):
            off = base + i * NBUF * CH_OUT
            for s in range(NBUF):
                process(s, off + s * CH_OUT, prefetch=True)
            return 0

        n_groups = n_chunks // NBUF
        lax.fori_loop(0, n_groups - 1, body, 0, unroll=False)
        tail = base + (n_groups - 1) * NBUF * CH_OUT
        for s in range(NBUF):
            process(s, tail + s * CH_OUT, prefetch=False)

    return gather_kernel


def kernel(x, table):
    S0, S1 = x.shape
    B = S0 * S1
    t2 = table.reshape(table.shape[0] // 2, 2 * table.shape[1])
    idx = x.reshape(B)
    out2 = _make_gather(B // 2)(t2, idx)
    return out2.reshape(S0, S1, EMB_DIM)


# speedup vs baseline: 1.7612x; 1.7612x over previous
import functools

import jax
import jax.numpy as jnp
from jax import lax
from jax.experimental import pallas as pl
from jax.experimental.pallas import tpu as pltpu
from jax.experimental.pallas import tpu_sc as plsc

EMB_DIM = 64
SCALE = 8.0

NUM_CORES = 2
NUM_SUBCORES = 16
NUM_WORKERS = NUM_CORES * NUM_SUBCORES

HALF = 64
NBUF = 4

NC = 4096
TB = 4


def _tc_transpose(table_t):
    d, v = table_t.shape

    def body(in_ref, out_ref):
        t = in_ref[...].T
        t3 = t.reshape(NC // 2, 2, d)
        out_ref[:, 0:d] = t3[:, 0, :]
        out_ref[:, d : 2 * d] = t3[:, 1, :]

    return pl.pallas_call(
        body,
        grid=(pl.cdiv(v, NC),),
        in_specs=[pl.BlockSpec((d, NC), lambda i: (0, i))],
        out_specs=pl.BlockSpec((NC // 2, 2 * d), lambda i: (i, 0)),
        out_shape=jax.ShapeDtypeStruct((v // 2, 2 * d), jnp.float32),
    )(table_t)


def _tc_pack(outg, s0, s1):
    hb = s0 // 2

    def body(in_ref, out_ref):
        for t in range(TB):
            sub = in_ref[t * hb : (t + 1) * hb, :]
            out_ref[t, :, 0:hb] = sub[:, 0:EMB_DIM].T * SCALE
            out_ref[t, :, hb : 2 * hb] = sub[:, EMB_DIM:128].T * SCALE

    return pl.pallas_call(
        body,
        grid=(s1 // TB,),
        in_specs=[pl.BlockSpec((TB * hb, 128), lambda i: (i, 0))],
        out_specs=pl.BlockSpec((TB, EMB_DIM, s0), lambda i: (i, 0, 0)),
        out_shape=jax.ShapeDtypeStruct((s1, EMB_DIM, s0), jnp.float32),
    )(outg)


def _make_gather(s0, s1):
    hb = s0 // 2
    n_chunks_total = s1 * hb // HALF
    assert n_chunks_total % (NUM_WORKERS * NBUF) == 0
    c_per_w = n_chunks_total // NUM_WORKERS
    cb_per_t = hb // HALF
    mesh = plsc.VectorSubcoreMesh(core_axis_name="c", subcore_axis_name="s")

    @functools.partial(
        pl.kernel,
        mesh=mesh,
        out_type=jax.ShapeDtypeStruct((s1 * hb, 128), jnp.float32),
        scratch_types=[
            pltpu.VMEM((NBUF, 2 * HALF), jnp.int32),
            pltpu.VMEM((NBUF, 2 * HALF, EMB_DIM), jnp.float32),
            pltpu.VMEM((NBUF, HALF, 128), jnp.float32),
            [pltpu.SemaphoreType.DMA] * NBUF,
            [pltpu.SemaphoreType.DMA] * NBUF,
        ],
        compiler_params=pltpu.CompilerParams(use_tc_tiling_on_sc=False),
    )
    def gather_kernel(table_hbm, xt_hbm, out_hbm, idx_v, g_v, o_v,
                      gsems, ssems):
        wid = lax.axis_index("s") * NUM_CORES + lax.axis_index("c")
        cbase = wid * c_per_w

        def start_gather(s, c):
            t = c // cb_per_t
            u0 = (c % cb_per_t) * HALF
            pltpu.sync_copy(xt_hbm.at[t, pl.ds(u0, HALF)],
                            idx_v.at[s].at[pl.ds(0, HALF)])
            pltpu.sync_copy(xt_hbm.at[t, pl.ds(hb + u0, HALF)],
                            idx_v.at[s].at[pl.ds(HALF, HALF)])
            pltpu.make_async_copy(
                table_hbm.at[idx_v.at[s]], g_v.at[s], gsems[s]
            ).start()

        def repack(s):
            def repack_body(j, _):
                for h in range(2):
                    row = g_v.at[s].at[HALF * h + j]
                    for k in range(EMB_DIM // 16):
                        o_v.at[s][j, pl.ds(EMB_DIM * h + 16 * k, 16)] = (
                            row[pl.ds(16 * k, 16)])
                return 0

            lax.fori_loop(0, HALF, repack_body, 0, unroll=False)

        def process(s, c, prefetch):
            pltpu.make_async_copy(
                table_hbm.at[idx_v.at[s]], g_v.at[s], gsems[s]
            ).wait()
            repack(s)
            t = c // cb_per_t
            u0 = (c % cb_per_t) * HALF
            dst = out_hbm.at[pl.ds(t * hb + u0, HALF)]
            pltpu.make_async_copy(o_v.at[s], dst, ssems[s]).start()
            pltpu.make_async_copy(o_v.at[s], dst, ssems[s]).wait()
            if prefetch:
                start_gather(s, c + NBUF)

        for s in range(NBUF):
            start_gather(s, cbase + s)

        def body(i, _):
            c = cbase + i * NBUF
            for s in range(NBUF):
                process(s, c + s, prefetch=True)
            return 0

        n_groups = c_per_w // NBUF
        lax.fori_loop(0, n_groups - 1, body, 0, unroll=False)
        tail = cbase + (n_groups - 1) * NBUF
        for s in range(NBUF):
            process(s, tail + s, prefetch=False)

    return gather_kernel


def kernel(x, table):
    S0, S1 = x.shape
    t2 = _tc_transpose(table.T)
    t4 = t2.reshape(table.shape[0], EMB_DIM)
    xt = x.T
    outg = _make_gather(S0, S1)(t4, xt)
    outp = _tc_pack(outg, S0, S1)
    return outp.transpose(2, 0, 1)


# speedup vs baseline: 1.9820x; 1.1254x over previous
import functools

import jax
import jax.numpy as jnp
from jax import lax
from jax.experimental import pallas as pl
from jax.experimental.pallas import tpu as pltpu
from jax.experimental.pallas import tpu_sc as plsc

EMB_DIM = 64
SCALE = 8.0

NUM_CORES = 2
NUM_SUBCORES = 16
NUM_WORKERS = NUM_CORES * NUM_SUBCORES

HALF = 64
NBUF = 4

NC = 4096
TB = 4


def _tc_transpose(table_t):
    d, v = table_t.shape

    def body(in_ref, out_ref):
        eye = jax.lax.broadcasted_iota(jnp.int32, (d, d), 0)
        eye = jnp.where(
            eye == jax.lax.broadcasted_iota(jnp.int32, (d, d), 1), 1.0, 0.0
        ).astype(jnp.float32)
        t = jax.lax.dot_general(
            in_ref[...], eye, (((0,), (0,)), ((), ())),
            preferred_element_type=jnp.float32)
        t3 = t.reshape(NC // 2, 2, d)
        out_ref[:, 0:d] = t3[:, 0, :]
        out_ref[:, d : 2 * d] = t3[:, 1, :]

    return pl.pallas_call(
        body,
        grid=(pl.cdiv(v, NC),),
        in_specs=[pl.BlockSpec((d, NC), lambda i: (0, i))],
        out_specs=pl.BlockSpec((NC // 2, 2 * d), lambda i: (i, 0)),
        out_shape=jax.ShapeDtypeStruct((v // 2, 2 * d), jnp.float32),
    )(table_t)


def _tc_pack(outg, s0, s1):
    hb = s0 // 2

    def body(in_ref, out_ref):
        for t in range(TB):
            sub = in_ref[t * hb : (t + 1) * hb, :]
            out_ref[t, :, 0:hb] = sub[:, 0:EMB_DIM].T * SCALE
            out_ref[t, :, hb : 2 * hb] = sub[:, EMB_DIM:128].T * SCALE

    return pl.pallas_call(
        body,
        grid=(s1 // TB,),
        in_specs=[pl.BlockSpec((TB * hb, 128), lambda i: (i, 0))],
        out_specs=pl.BlockSpec((TB, EMB_DIM, s0), lambda i: (i, 0, 0)),
        out_shape=jax.ShapeDtypeStruct((s1, EMB_DIM, s0), jnp.float32),
    )(outg)


def _make_gather(s0, s1):
    hb = s0 // 2
    assert hb == HALF * NUM_WORKERS and s1 % NBUF == 0
    mesh = plsc.VectorSubcoreMesh(core_axis_name="c", subcore_axis_name="s")

    @functools.partial(
        pl.kernel,
        mesh=mesh,
        out_type=jax.ShapeDtypeStruct((s1 * hb, 128), jnp.float32),
        scratch_types=[
            pltpu.VMEM((s1, HALF), jnp.int32),
            pltpu.VMEM((s1, HALF), jnp.int32),
            pltpu.VMEM((NBUF, 2 * HALF, EMB_DIM), jnp.float32),
            pltpu.VMEM((NBUF, HALF, 128), jnp.float32),
            [pltpu.SemaphoreType.DMA] * NBUF,
            [pltpu.SemaphoreType.DMA] * NBUF,
        ],
        compiler_params=pltpu.CompilerParams(use_tc_tiling_on_sc=False),
    )
    def gather_kernel(table_hbm, xt_hbm, out_hbm, ilo_v, ihi_v, g_v, o_v,
                      gsems, ssems):
        wid = lax.axis_index("s") * NUM_CORES + lax.axis_index("c")
        u0 = wid * HALF

        pltpu.sync_copy(xt_hbm.at[:, pl.ds(u0, HALF)], ilo_v)
        pltpu.sync_copy(xt_hbm.at[:, pl.ds(hb + u0, HALF)], ihi_v)

        def start_gather(s, t):
            pltpu.make_async_copy(
                table_hbm.at[ilo_v.at[t]],
                g_v.at[s].at[pl.ds(0, HALF)], gsems[s],
            ).start()
            pltpu.make_async_copy(
                table_hbm.at[ihi_v.at[t]],
                g_v.at[s].at[pl.ds(HALF, HALF)], gsems[s],
            ).start()

        def repack(s):
            def repack_body(j, _):
                for h in range(2):
                    row = g_v.at[s].at[HALF * h + j]
                    for k in range(EMB_DIM // 16):
                        o_v.at[s][j, pl.ds(EMB_DIM * h + 16 * k, 16)] = (
                            row[pl.ds(16 * k, 16)])
                return 0

            lax.fori_loop(0, HALF, repack_body, 0, unroll=False)

        def process(s, t, prefetch):
            pltpu.make_async_copy(
                table_hbm.at[pl.ds(0, 2 * HALF)], g_v.at[s], gsems[s]
            ).wait()
            repack(s)
            dst = out_hbm.at[pl.ds(t * hb + u0, HALF)]
            pltpu.make_async_copy(o_v.at[s], dst, ssems[s]).start()
            pltpu.make_async_copy(o_v.at[s], dst, ssems[s]).wait()
            if prefetch:
                start_gather(s, t + NBUF)

        for s in range(NBUF):
            start_gather(s, s)

        def body(i, _):
            t = i * NBUF
            for s in range(NBUF):
                process(s, t + s, prefetch=True)
            return 0

        n_groups = s1 // NBUF
        lax.fori_loop(0, n_groups - 1, body, 0, unroll=False)
        tail = (n_groups - 1) * NBUF
        for s in range(NBUF):
            process(s, tail + s, prefetch=False)

    return gather_kernel


def kernel(x, table):
    S0, S1 = x.shape
    t2 = _tc_transpose(table.T)
    t4 = t2.reshape(table.shape[0], EMB_DIM)
    xt = x.T
    outg = _make_gather(S0, S1)(t4, xt)
    outp = _tc_pack(outg, S0, S1)
    return outp.transpose(2, 0, 1)


# speedup vs baseline: 2.9463x; 1.4865x over previous
import functools

import jax
import jax.numpy as jnp
from jax import lax
from jax.experimental import pallas as pl
from jax.experimental.pallas import tpu as pltpu
from jax.experimental.pallas import tpu_sc as plsc

EMB_DIM = 64
SCALE = 8.0

NUM_CORES = 2
NUM_SUBCORES = 16
NUM_WORKERS = NUM_CORES * NUM_SUBCORES

HALF = 64
NBUF = 8

NC = 8192
TB = 4


def _tc_transpose(table_t):
    d, v = table_t.shape

    def body(in_ref, out_ref):
        t = in_ref[...].T
        t3 = t.reshape(NC // 2, 2, d)
        out_ref[:, 0:d] = t3[:, 0, :]
        out_ref[:, d : 2 * d] = t3[:, 1, :]

    return pl.pallas_call(
        body,
        grid=(pl.cdiv(v, NC),),
        in_specs=[pl.BlockSpec((d, NC), lambda i: (0, i))],
        out_specs=pl.BlockSpec((NC // 2, 2 * d), lambda i: (i, 0)),
        out_shape=jax.ShapeDtypeStruct((v // 2, 2 * d), jnp.float32),
    )(table_t)


def _tc_pack(outg, s0, s1):
    hb = s0 // 2

    def body(in_ref, out_ref):
        for t in range(TB):
            sub = in_ref[t * hb : (t + 1) * hb, :]
            out_ref[t, :, 0:hb] = sub[:, 0:EMB_DIM].T * SCALE
            out_ref[t, :, hb : 2 * hb] = sub[:, EMB_DIM:128].T * SCALE

    return pl.pallas_call(
        body,
        grid=(s1 // TB,),
        in_specs=[pl.BlockSpec((TB * hb, 128), lambda i: (i, 0))],
        out_specs=pl.BlockSpec((TB, EMB_DIM, s0), lambda i: (i, 0, 0)),
        out_shape=jax.ShapeDtypeStruct((s1, EMB_DIM, s0), jnp.float32),
    )(outg)


def _make_gather(s0, s1):
    hb = s0 // 2
    assert hb == HALF * NUM_WORKERS and s1 % NBUF == 0
    mesh = plsc.VectorSubcoreMesh(core_axis_name="c", subcore_axis_name="s")

    @functools.partial(
        pl.kernel,
        mesh=mesh,
        out_type=jax.ShapeDtypeStruct((s0 * s1, EMB_DIM), jnp.float32),
        scratch_types=[
            pltpu.VMEM((s1, 2, HALF), jnp.int32),
            pltpu.VMEM((NBUF, 2 * HALF), jnp.int32),
            pltpu.VMEM((NBUF, 2 * HALF, EMB_DIM), jnp.float32),
            [pltpu.SemaphoreType.DMA] * NBUF,
            [pltpu.SemaphoreType.DMA] * NBUF,
        ],
        compiler_params=pltpu.CompilerParams(
            use_tc_tiling_on_sc=False, needs_layout_passes=False),
    )
    def gather_kernel(table_hbm, xt_hbm, out_hbm, ibuf, idx_v, g_v,
                      gsems, ssems):
        wid = lax.axis_index("s") * NUM_CORES + lax.axis_index("c")
        u0 = wid * HALF

        pltpu.sync_copy(xt_hbm.at[:, pl.ds(u0, HALF)], ibuf.at[:, 0])
        pltpu.sync_copy(xt_hbm.at[:, pl.ds(hb + u0, HALF)], ibuf.at[:, 1])

        lanes = lax.iota(jnp.int32, 16)
        h_vec = lanes & 1
        j_half = lax.shift_right_logical(lanes, 1)

        def start_gather(s, t):
            t_vec = jnp.full((16,), 0, jnp.int32) + t
            for k in range(2 * HALF // 16):
                idx_v.at[s][pl.ds(16 * k, 16)] = plsc.load_gather(
                    ibuf, [t_vec, h_vec, j_half + 8 * k])
            pltpu.make_async_copy(
                table_hbm.at[idx_v.at[s]], g_v.at[s], gsems[s]
            ).start()

        def process(s, t, prefetch):
            pltpu.make_async_copy(
                table_hbm.at[idx_v.at[s]], g_v.at[s], gsems[s]
            ).wait()
            dst = out_hbm.at[pl.ds(2 * (t * hb + u0), 2 * HALF)]
            pltpu.make_async_copy(g_v.at[s], dst, ssems[s]).start()
            pltpu.make_async_copy(g_v.at[s], dst, ssems[s]).wait()
            if prefetch:
                start_gather(s, t + NBUF)

        for s in range(NBUF):
            start_gather(s, s)

        def body(i, _):
            t = i * NBUF
            for s in range(NBUF):
                process(s, t + s, prefetch=True)
            return 0

        n_groups = s1 // NBUF
        lax.fori_loop(0, n_groups - 1, body, 0, unroll=False)
        tail = (n_groups - 1) * NBUF
        for s in range(NBUF):
            process(s, tail + s, prefetch=False)

    return gather_kernel


def kernel(x, table):
    S0, S1 = x.shape
    t2 = _tc_transpose(table.T)
    t4 = t2.reshape(table.shape[0], EMB_DIM)
    xt = x.T
    outg = _make_gather(S0, S1)(t4, xt)
    outp = _tc_pack(outg.reshape(S0 * S1 // 2, 128), S0, S1)
    return outp.transpose(2, 0, 1)
